# baseline (speedup 1.0000x reference)
import functools

import jax
import jax.numpy as jnp
from jax import lax
from jax.experimental import pallas as pl
from jax.experimental.pallas import tpu as pltpu
from jax.experimental.pallas import tpu_sc as plsc

H = 8
NC = 2
NS = 16
NW = NC * NS
CH = 128
ALIGN = NW * 1024


def _pick_bm(m, cap=2048):
    for bm in range(min(cap, m), 0, -8):
        if m % bm == 0:
            return bm
    return m


def _kb(d):
    return 4 if d > 64 else 8


def _mm(x, w, b=None):
    m, k = x.shape
    n = w.shape[0]
    wt = w.T
    if b is None:
        b = jnp.zeros((n,), jnp.float32)
    b2 = b.reshape(1, n)
    bm = _pick_bm(m)

    def body(x_ref, w_ref, b_ref, o_ref):
        o_ref[...] = (
            jnp.dot(x_ref[...], w_ref[...], preferred_element_type=jnp.float32)
            + b_ref[...]
        )

    return pl.pallas_call(
        body,
        grid=(m // bm,),
        in_specs=[
            pl.BlockSpec((bm, k), lambda i: (i, 0)),
            pl.BlockSpec((k, n), lambda i: (0, 0)),
            pl.BlockSpec((1, n), lambda i: (0, 0)),
        ],
        out_specs=pl.BlockSpec((bm, n), lambda i: (i, 0)),
        out_shape=jax.ShapeDtypeStruct((m, n), jnp.float32),
    )(x, wt, b2)


def _edge_logits(xs, xd, ef_raw, w_cat, bias, attn_mat, sum_mat, oe, n_real):
    ep, k = xs.shape
    e_raw, ke = ef_raw.shape
    w = w_cat.shape[1]
    import math
    bm = _pick_bm(math.gcd(ep, e_raw), cap=4096)
    nraw_blocks = e_raw // bm

    def ef_map(i):
        return (jnp.minimum(i, nraw_blocks - 1), 0)

    def body(xs_ref, xd_ref, ef_ref, w_ref, b_ref, am_ref, sm_ref,
             efo_ref, ee_ref):
        i = pl.program_id(0)
        x = jnp.concatenate([xs_ref[...], xd_ref[...], ef_ref[...]], axis=1)
        s = jnp.dot(x, w_ref[...], preferred_element_type=jnp.float32) + b_ref[...]
        s = jnp.where(s >= 0.0, s, 0.01 * s)
        efo_ref[...] = jnp.dot(s, sm_ref[...], preferred_element_type=jnp.float32)
        e16 = jnp.dot(s, am_ref[...], preferred_element_type=jnp.float32)
        ee = jnp.exp(e16)
        lane = lax.broadcasted_iota(jnp.int32, (bm, 16), 1)
        rid = i * bm + lax.broadcasted_iota(jnp.int32, (bm, 16), 0)
        ee = jnp.where((rid < n_real) & (lane < H), ee, 0.0)
        ee_ref[...] = ee

    kc = 2 * k + ke
    return pl.pallas_call(
        body,
        grid=(ep // bm,),
        in_specs=[
            pl.BlockSpec((bm, k), lambda i: (i, 0)),
            pl.BlockSpec((bm, k), lambda i: (i, 0)),
            pl.BlockSpec((bm, ke), ef_map),
            pl.BlockSpec((kc, w), lambda i: (0, 0)),
            pl.BlockSpec((1, w), lambda i: (0, 0)),
            pl.BlockSpec((w, 16), lambda i: (0, 0)),
            pl.BlockSpec((w, oe), lambda i: (0, 0)),
        ],
        out_specs=[
            pl.BlockSpec((bm, oe), lambda i: (i, 0)),
            pl.BlockSpec((bm, 16), lambda i: (i, 0)),
        ],
        out_shape=[
            jax.ShapeDtypeStruct((ep, oe), jnp.float32),
            jax.ShapeDtypeStruct((ep, 16), jnp.float32),
        ],
    )(xs, xd, ef_raw, w_cat, bias.reshape(1, w), attn_mat, sum_mat)


def _edge_messages(xs, ee, gd, wnode_t, bnode, expand_mat, sum_mat, on):
    ep, k = xs.shape
    w = H * on
    sub = on // 16
    bm = _pick_bm(ep, cap=2048 if w > 128 else 4096)

    def body(xs_ref, ee_ref, gd_ref, wn_ref, bn_ref, em_ref, sm_ref, msg_ref):
        hh = (
            jnp.dot(xs_ref[...], wn_ref[...], preferred_element_type=jnp.float32)
            + bn_ref[...]
        )
        gd = gd_ref[...]
        a = jnp.where(gd > 0.0, ee_ref[...] / gd, 0.0)
        aw = jnp.dot(a, em_ref[...], preferred_element_type=jnp.float32)
        msg_ref[...] = jnp.dot(aw * hh, sm_ref[...],
                               preferred_element_type=jnp.float32)

    return pl.pallas_call(
        body,
        grid=(ep // bm,),
        in_specs=[
            pl.BlockSpec((bm, k), lambda i: (i, 0)),
            pl.BlockSpec((bm, 16), lambda i: (i, 0)),
            pl.BlockSpec((bm, 16), lambda i: (i, 0)),
            pl.BlockSpec((k, w), lambda i: (0, 0)),
            pl.BlockSpec((1, w), lambda i: (0, 0)),
            pl.BlockSpec((16, w), lambda i: (0, 0)),
            pl.BlockSpec((w, on), lambda i: (0, 0)),
        ],
        out_specs=pl.BlockSpec((bm, on), lambda i: (i, 0)),
        out_shape=jax.ShapeDtypeStruct((ep, on), jnp.float32),
    )(xs, ee, gd, wnode_t, bnode.reshape(1, w), expand_mat, sum_mat)


def _add2(a, b):
    m, w = a.shape
    bm = _pick_bm(m)

    def body(a_ref, b_ref, o_ref):
        o_ref[...] = a_ref[...] + b_ref[...]

    return pl.pallas_call(
        body,
        grid=(m // bm,),
        in_specs=[
            pl.BlockSpec((bm, w), lambda i: (i, 0)),
            pl.BlockSpec((bm, w), lambda i: (i, 0)),
        ],
        out_specs=pl.BlockSpec((bm, w), lambda i: (i, 0)),
        out_shape=jax.ShapeDtypeStruct((m, w), jnp.float32),
    )(a, b)


def _sc_mesh():
    return plsc.VectorSubcoreMesh(core_axis_name="c", subcore_axis_name="s")


def _gather2(table, src, dst):
    v, d = table.shape
    b = src.shape[0]
    kb = _kb(d)
    blk = kb * CH
    per_w = b // NW
    assert per_w % blk == 0, (b, d)
    n_blocks = per_w // blk
    cp = None if d % 128 == 0 else pltpu.CompilerParams(use_tc_tiling_on_sc=False)
    src2 = src.reshape(-1, CH)
    dst2 = dst.reshape(-1, CH)
    sds = jax.ShapeDtypeStruct((b // CH, CH, d), jnp.float32)

    @functools.partial(
        pl.kernel,
        out_type=(sds, sds),
        mesh=_sc_mesh(),
        scratch_types=[
            pltpu.VMEM((kb, CH), jnp.int32),
            pltpu.VMEM((kb, CH, d), jnp.float32),
            pltpu.SemaphoreType.DMA,
        ],
        compiler_params=cp,
    )
    def k(tab_hbm, src_hbm, dst_hbm, xs_hbm, xd_hbm, idx_v, rows_v, sem):
        wid = lax.axis_index("s") * NC + lax.axis_index("c")
        base = wid * (per_w // CH)

        def one(idx_hbm, out_hbm, g, carry):
            r0 = pl.multiple_of(base + g * kb, kb)
            pltpu.sync_copy(idx_hbm.at[pl.ds(r0, kb)], idx_v)
            cps = [
                pltpu.async_copy(tab_hbm.at[idx_v.at[j]], rows_v.at[j], sem)
                for j in range(kb)
            ]
            for c in cps:
                c.wait()
            pltpu.sync_copy(rows_v, out_hbm.at[pl.ds(r0, kb)])
            return carry

        lax.fori_loop(0, n_blocks, functools.partial(one, src_hbm, xs_hbm), 0)
        lax.fori_loop(0, n_blocks, functools.partial(one, dst_hbm, xd_hbm), 0)

    xs, xd = k(table, src2, dst2)
    return xs.reshape(b, d), xd.reshape(b, d)


def _gather1(table, idx):
    v, d = table.shape
    b = idx.shape[0]
    kb = _kb(d)
    blk = kb * CH
    per_w = b // NW
    assert per_w % blk == 0, (b, d)
    n_blocks = per_w // blk
    cp = None if d % 128 == 0 else pltpu.CompilerParams(use_tc_tiling_on_sc=False)
    idx2 = idx.reshape(-1, CH)

    @functools.partial(
        pl.kernel,
        out_type=jax.ShapeDtypeStruct((b // CH, CH, d), jnp.float32),
        mesh=_sc_mesh(),
        scratch_types=[
            pltpu.VMEM((kb, CH), jnp.int32),
            pltpu.VMEM((kb, CH, d), jnp.float32),
            pltpu.SemaphoreType.DMA,
        ],
        compiler_params=cp,
    )
    def k(tab_hbm, idx_hbm, out_hbm, idx_v, rows_v, sem):
        wid = lax.axis_index("s") * NC + lax.axis_index("c")
        base = wid * (per_w // CH)

        def one(g, carry):
            r0 = pl.multiple_of(base + g * kb, kb)
            pltpu.sync_copy(idx_hbm.at[pl.ds(r0, kb)], idx_v)
            cps = [
                pltpu.async_copy(tab_hbm.at[idx_v.at[j]], rows_v.at[j], sem)
                for j in range(kb)
            ]
            for c in cps:
                c.wait()
            pltpu.sync_copy(rows_v, out_hbm.at[pl.ds(r0, kb)])
            return carry

        lax.fori_loop(0, n_blocks, one, 0)

    return k(table, idx2).reshape(b, d)


def _softmax_denom(ee, dst, n_out, zeros):
    b, w = ee.shape
    kb = _kb(w)
    blk = kb * CH
    per_core = b // NS
    per_w = b // NW
    assert per_core % blk == 0 and per_w % blk == 0
    n2 = per_core // blk
    n3 = per_w // blk
    dst2 = dst.reshape(-1, CH)
    ee3 = ee.reshape(-1, CH, w)
    cp = None if w % 128 == 0 else pltpu.CompilerParams(use_tc_tiling_on_sc=False)

    @functools.partial(
        pl.kernel,
        out_type=jax.ShapeDtypeStruct((b // CH, CH, w), jnp.float32),
        mesh=_sc_mesh(),
        scratch_types=[
            pltpu.VMEM((kb, CH), jnp.int32),
            pltpu.VMEM((kb, CH, w), jnp.float32),
            pltpu.VMEM_SHARED((n_out, w), jnp.float32),
            pltpu.SemaphoreType.DMA,
        ],
        compiler_params=cp,
    )
    def k(ee_hbm, dst_hbm, zero_hbm, gd_hbm, idx_v, val_v, acc_sh, sem):
        sid = lax.axis_index("s")
        cid = lax.axis_index("c")

        @pl.when(sid == 0)
        def _():
            pltpu.sync_copy(zero_hbm, acc_sh)

        plsc.subcore_barrier()

        def accum(g, carry):
            r0 = pl.multiple_of(sid * (per_core // CH) + g * kb, kb)
            pltpu.sync_copy(dst_hbm.at[pl.ds(r0, kb)], idx_v)
            pltpu.sync_copy(ee_hbm.at[pl.ds(r0, kb)], val_v)
            for j in range(kb):
                pltpu.sync_copy(val_v.at[j], acc_sh.at[idx_v.at[j]], add=True)
            return carry

        lax.fori_loop(0, n2, accum, 0)
        plsc.subcore_barrier()
        wid = sid * NC + cid

        def readout(g, carry):
            r0 = pl.multiple_of(wid * (per_w // CH) + g * kb, kb)
            pltpu.sync_copy(dst_hbm.at[pl.ds(r0, kb)], idx_v)
            cps = [
                pltpu.async_copy(acc_sh.at[idx_v.at[j]], val_v.at[j], sem)
                for j in range(kb)
            ]
            for c in cps:
                c.wait()
            pltpu.sync_copy(val_v, gd_hbm.at[pl.ds(r0, kb)])
            return carry

        lax.fori_loop(0, n3, readout, 0)

    return k(ee3, dst2, zeros).reshape(b, w)


def _scatter_add(values, idx, n_out, zeros):
    b, w = values.shape
    kb = 1 if w >= 128 else _kb(w)
    blk = kb * CH
    per_w = b // NW
    assert per_w % blk == 0
    n_blocks = per_w // blk
    rows_per_tile = (n_out // 8 // NS) * 8
    rows_rem = n_out - rows_per_tile * NS
    idx2 = idx.reshape(-1, CH)
    val3 = values.reshape(-1, CH, w)
    cp = None if w % 128 == 0 else pltpu.CompilerParams(use_tc_tiling_on_sc=False)

    @functools.partial(
        pl.kernel,
        out_type=jax.ShapeDtypeStruct((2, n_out, w), jnp.float32),
        mesh=_sc_mesh(),
        scratch_types=[
            pltpu.VMEM((kb, CH), jnp.int32),
            pltpu.VMEM((kb, CH, w), jnp.float32),
            pltpu.VMEM_SHARED((n_out, w), jnp.float32),
        ],
        compiler_params=cp,
    )
    def k(val_hbm, idx_hbm, zero_hbm, out_hbm, idx_v, val_v, acc_sh):
        cid = lax.axis_index("c")
        sid = lax.axis_index("s")
        wid = sid * NC + cid

        @pl.when(sid == 0)
        def _():
            pltpu.sync_copy(zero_hbm, acc_sh)

        plsc.subcore_barrier()

        def body(g, carry):
            r0 = pl.multiple_of(wid * (per_w // CH) + g * kb, kb)
            pltpu.sync_copy(idx_hbm.at[pl.ds(r0, kb)], idx_v)
            pltpu.sync_copy(val_hbm.at[pl.ds(r0, kb)], val_v)
            for j in range(kb):
                pltpu.sync_copy(val_v.at[j], acc_sh.at[idx_v.at[j]], add=True)
            return carry

        lax.fori_loop(0, n_blocks, body, 0)
        plsc.subcore_barrier()
        r0 = pl.multiple_of(sid * rows_per_tile, 8)
        pltpu.sync_copy(
            acc_sh.at[pl.ds(r0, rows_per_tile)],
            out_hbm.at[cid, pl.ds(r0, rows_per_tile)],
        )
        if rows_rem:
            @pl.when(sid == 0)
            def _():
                rr = rows_per_tile * NS
                pltpu.sync_copy(
                    acc_sh.at[pl.ds(rr, rows_rem)],
                    out_hbm.at[cid, pl.ds(rr, rows_rem)],
                )

    parts = k(val3, idx2, zeros)
    return _add2(parts[0], parts[1])


def _head_mats(oe, on, attn):
    eye_e = jnp.eye(oe, dtype=jnp.float32)
    sum_e = jnp.tile(eye_e, (H, 1))
    attn_mat = jnp.zeros((H * oe, 16), jnp.float32)
    for h in range(H):
        attn_mat = attn_mat.at[h * oe:(h + 1) * oe, h].set(attn[0, h, :])
    expand = jnp.zeros((16, H * on), jnp.float32)
    for h in range(H):
        expand = expand.at[h, h * on:(h + 1) * on].set(1.0)
    sum_n = jnp.tile(jnp.eye(on, dtype=jnp.float32), (H, 1))
    return attn_mat, sum_e, expand, sum_n


def _egat(nfeats, efeats_raw, src_p, dst_p, num_nodes, p, out_n, out_e, n_real,
          need_nf=True):
    xs, xd = _gather2(nfeats, src_p, dst_p)
    attn_mat, sum_e, expand, sum_n = _head_mats(out_e, out_n, p["attn"])
    w_cat = jnp.concatenate([p["Wni"].T, p["Wnj"].T, p["Wfij"].T], axis=0)
    ef, ee = _edge_logits(
        xs, xd, efeats_raw, w_cat, p["bias"], attn_mat, sum_e, out_e, n_real,
    )
    if not need_nf:
        return None, ef
    zeros16 = jnp.zeros((num_nodes, 16), jnp.float32)
    den = _scatter_add(ee, dst_p, num_nodes, zeros16)
    gd = _gather1(den, dst_p)
    msg = _edge_messages(xs, ee, gd, p["Wnode"].T, p["bnode"], expand, sum_n,
                         out_n)
    zeros_n = jnp.zeros((num_nodes, out_n), jnp.float32)
    nf = _scatter_add(msg, dst_p, num_nodes, zeros_n)
    return nf, ef


def _pad_edges(x, ep):
    e = x.shape[0]
    if e == ep:
        return x
    pad = [(0, ep - e)] + [(0, 0)] * (x.ndim - 1)
    return jnp.pad(x, pad)


def _mm_pairs(ef_p, w, b, n_rows):
    ke = ef_p.shape[1]
    n = w.shape[0]
    wt = w.T
    bm = _pick_bm(n_rows, cap=4096)

    x11 = ef_p.reshape(-1, 2 * ke)

    def body(x_ref, w_ref, b_ref, o_ref):
        o_ref[...] = (
            jnp.dot(x_ref[...], w_ref[...], preferred_element_type=jnp.float32)
            + b_ref[...]
        )

    return pl.pallas_call(
        body,
        grid=(n_rows // bm,),
        in_specs=[
            pl.BlockSpec((bm, 2 * ke), lambda i: (i, 0)),
            pl.BlockSpec((2 * ke, n), lambda i: (0, 0)),
            pl.BlockSpec((1, n), lambda i: (0, 0)),
        ],
        out_specs=pl.BlockSpec((bm, n), lambda i: (i, 0)),
        out_shape=jax.ShapeDtypeStruct((n_rows, n), jnp.float32),
    )(x11, wt, b.reshape(1, n))


def kernel(node_feats, edge_feats, node_path, edge_path, params,
           edge_index_lg, edge_index_gg):
    del node_path
    n1, e1 = node_feats.shape[0], edge_feats.shape[0]
    n2, e2 = e1 // 2, edge_path.shape[0]
    e1p = ((e1 + ALIGN - 1) // ALIGN) * ALIGN
    e2p = ((e2 + ALIGN - 1) // ALIGN) * ALIGN

    src1 = _pad_edges(edge_index_lg[0], e1p)
    dst1 = _pad_edges(edge_index_lg[1], e1p)
    src2 = _pad_edges(edge_index_gg[0], e2p)
    dst2 = _pad_edges(edge_index_gg[1], e2p)

    p1, p2 = params["gcn3"]
    nf1, ef = _egat(node_feats, edge_feats, src1, dst1, n1, p1, 128, 16, e1)
    _, ef = _egat(nf1, ef, src1, dst1, n1, p2, 128, 16, e1, need_nf=False)

    npth = _mm_pairs(ef, params["Wlin"], params["blin"], n2)

    h3, f3 = _egat(npth, edge_path, src2, dst2, n2, params["l4"], 16, 16, e2)
    return h3, f3[:e2]


# speedup vs baseline: 1.0449x; 1.0449x over previous
import functools

import jax
import jax.numpy as jnp
from jax import lax
from jax.experimental import pallas as pl
from jax.experimental.pallas import tpu as pltpu
from jax.experimental.pallas import tpu_sc as plsc

H = 8
NC = 2
NS = 16
NW = NC * NS
CH = 128
ALIGN = NW * 1024


def _pick_bm(m, cap=2048):
    for bm in range(min(cap, m), 0, -8):
        if m % bm == 0:
            return bm
    return m


def _kb(d):
    return 4 if d > 64 else 8


def _mm(x, w, b=None):
    m, k = x.shape
    n = w.shape[0]
    wt = w.T
    if b is None:
        b = jnp.zeros((n,), jnp.float32)
    b2 = b.reshape(1, n)
    bm = _pick_bm(m)

    def body(x_ref, w_ref, b_ref, o_ref):
        o_ref[...] = (
            jnp.dot(x_ref[...], w_ref[...], preferred_element_type=jnp.float32)
            + b_ref[...]
        )

    return pl.pallas_call(
        body,
        grid=(m // bm,),
        in_specs=[
            pl.BlockSpec((bm, k), lambda i: (i, 0)),
            pl.BlockSpec((k, n), lambda i: (0, 0)),
            pl.BlockSpec((1, n), lambda i: (0, 0)),
        ],
        out_specs=pl.BlockSpec((bm, n), lambda i: (i, 0)),
        out_shape=jax.ShapeDtypeStruct((m, n), jnp.float32),
    )(x, wt, b2)


def _edge_logits(xs, xd, ef_raw, w_cat, bias, attn_mat, sum_mat, oe, n_real):
    ep, k = xs.shape
    e_raw, ke = ef_raw.shape
    w = w_cat.shape[1]
    import math
    bm = _pick_bm(math.gcd(ep, e_raw), cap=4096)
    nraw_blocks = e_raw // bm

    def ef_map(i):
        return (jnp.minimum(i, nraw_blocks - 1), 0)

    def body(xs_ref, xd_ref, ef_ref, w_ref, b_ref, am_ref, sm_ref,
             efo_ref, ee_ref):
        i = pl.program_id(0)
        x = jnp.concatenate([xs_ref[...], xd_ref[...], ef_ref[...]], axis=1)
        s = jnp.dot(x, w_ref[...], preferred_element_type=jnp.float32) + b_ref[...]
        s = jnp.where(s >= 0.0, s, 0.01 * s)
        efo_ref[...] = jnp.dot(s, sm_ref[...], preferred_element_type=jnp.float32)
        e16 = jnp.dot(s, am_ref[...], preferred_element_type=jnp.float32)
        ee = jnp.exp(e16)
        lane = lax.broadcasted_iota(jnp.int32, (bm, 16), 1)
        rid = i * bm + lax.broadcasted_iota(jnp.int32, (bm, 16), 0)
        ee = jnp.where((rid < n_real) & (lane < H), ee, 0.0)
        ee_ref[...] = ee

    kc = 2 * k + ke
    return pl.pallas_call(
        body,
        grid=(ep // bm,),
        in_specs=[
            pl.BlockSpec((bm, k), lambda i: (i, 0)),
            pl.BlockSpec((bm, k), lambda i: (i, 0)),
            pl.BlockSpec((bm, ke), ef_map),
            pl.BlockSpec((kc, w), lambda i: (0, 0)),
            pl.BlockSpec((1, w), lambda i: (0, 0)),
            pl.BlockSpec((w, 16), lambda i: (0, 0)),
            pl.BlockSpec((w, oe), lambda i: (0, 0)),
        ],
        out_specs=[
            pl.BlockSpec((bm, oe), lambda i: (i, 0)),
            pl.BlockSpec((bm, 16), lambda i: (i, 0)),
        ],
        out_shape=[
            jax.ShapeDtypeStruct((ep, oe), jnp.float32),
            jax.ShapeDtypeStruct((ep, 16), jnp.float32),
        ],
    )(xs, xd, ef_raw, w_cat, bias.reshape(1, w), attn_mat, sum_mat)


def _edge_messages(xs, ee, gd, wnode_t, bnode, expand_mat, sum_mat, on):
    ep, k = xs.shape
    w = H * on
    sub = on // 16
    bm = _pick_bm(ep, cap=2048 if w > 128 else 4096)

    def body(xs_ref, ee_ref, gd_ref, wn_ref, bn_ref, em_ref, sm_ref, msg_ref):
        hh = (
            jnp.dot(xs_ref[...], wn_ref[...], preferred_element_type=jnp.float32)
            + bn_ref[...]
        )
        gd = gd_ref[...]
        a = jnp.where(gd > 0.0, ee_ref[...] / gd, 0.0)
        aw = jnp.dot(a, em_ref[...], preferred_element_type=jnp.float32)
        msg_ref[...] = jnp.dot(aw * hh, sm_ref[...],
                               preferred_element_type=jnp.float32)

    return pl.pallas_call(
        body,
        grid=(ep // bm,),
        in_specs=[
            pl.BlockSpec((bm, k), lambda i: (i, 0)),
            pl.BlockSpec((bm, 16), lambda i: (i, 0)),
            pl.BlockSpec((bm, 16), lambda i: (i, 0)),
            pl.BlockSpec((k, w), lambda i: (0, 0)),
            pl.BlockSpec((1, w), lambda i: (0, 0)),
            pl.BlockSpec((16, w), lambda i: (0, 0)),
            pl.BlockSpec((w, on), lambda i: (0, 0)),
        ],
        out_specs=pl.BlockSpec((bm, on), lambda i: (i, 0)),
        out_shape=jax.ShapeDtypeStruct((ep, on), jnp.float32),
    )(xs, ee, gd, wnode_t, bnode.reshape(1, w), expand_mat, sum_mat)


def _add2(a, b):
    m, w = a.shape
    bm = _pick_bm(m)

    def body(a_ref, b_ref, o_ref):
        o_ref[...] = a_ref[...] + b_ref[...]

    return pl.pallas_call(
        body,
        grid=(m // bm,),
        in_specs=[
            pl.BlockSpec((bm, w), lambda i: (i, 0)),
            pl.BlockSpec((bm, w), lambda i: (i, 0)),
        ],
        out_specs=pl.BlockSpec((bm, w), lambda i: (i, 0)),
        out_shape=jax.ShapeDtypeStruct((m, w), jnp.float32),
    )(a, b)


def _sc_mesh():
    return plsc.VectorSubcoreMesh(core_axis_name="c", subcore_axis_name="s")


def _gather2(table, src, dst):
    v, d = table.shape
    b = src.shape[0]
    kb = _kb(d)
    blk = kb * CH
    per_w = b // NW
    assert per_w % blk == 0, (b, d)
    n_blocks = per_w // blk
    cp = None if d % 128 == 0 else pltpu.CompilerParams(use_tc_tiling_on_sc=False)
    src2 = src.reshape(-1, CH)
    dst2 = dst.reshape(-1, CH)
    sds = jax.ShapeDtypeStruct((b // CH, CH, d), jnp.float32)

    @functools.partial(
        pl.kernel,
        out_type=(sds, sds),
        mesh=_sc_mesh(),
        scratch_types=[
            pltpu.VMEM((kb, CH), jnp.int32),
            pltpu.VMEM((kb, CH, d), jnp.float32),
            pltpu.SemaphoreType.DMA,
        ],
        compiler_params=cp,
    )
    def k(tab_hbm, src_hbm, dst_hbm, xs_hbm, xd_hbm, idx_v, rows_v, sem):
        wid = lax.axis_index("s") * NC + lax.axis_index("c")
        base = wid * (per_w // CH)

        def one(idx_hbm, out_hbm, g, carry):
            r0 = pl.multiple_of(base + g * kb, kb)
            pltpu.sync_copy(idx_hbm.at[pl.ds(r0, kb)], idx_v)
            cps = [
                pltpu.async_copy(tab_hbm.at[idx_v.at[j]], rows_v.at[j], sem)
                for j in range(kb)
            ]
            for c in cps:
                c.wait()
            pltpu.sync_copy(rows_v, out_hbm.at[pl.ds(r0, kb)])
            return carry

        lax.fori_loop(0, n_blocks, functools.partial(one, src_hbm, xs_hbm), 0)
        lax.fori_loop(0, n_blocks, functools.partial(one, dst_hbm, xd_hbm), 0)

    xs, xd = k(table, src2, dst2)
    return xs.reshape(b, d), xd.reshape(b, d)


def _softmax_denom(ee, dst, n_out, zeros):
    b, w = ee.shape
    kb = _kb(w)
    blk = kb * CH
    per_core = b // NS
    per_w = b // NW
    assert per_core % blk == 0 and per_w % blk == 0
    n2 = per_core // blk
    n3 = per_w // blk
    dst2 = dst.reshape(-1, CH)
    ee3 = ee.reshape(-1, CH, w)
    cp = None if w % 128 == 0 else pltpu.CompilerParams(use_tc_tiling_on_sc=False)

    @functools.partial(
        pl.kernel,
        out_type=jax.ShapeDtypeStruct((b // CH, CH, w), jnp.float32),
        mesh=_sc_mesh(),
        scratch_types=[
            pltpu.VMEM((kb, CH), jnp.int32),
            pltpu.VMEM((kb, CH, w), jnp.float32),
            pltpu.VMEM_SHARED((n_out, w), jnp.float32),
            pltpu.SemaphoreType.DMA,
        ],
        compiler_params=cp,
    )
    def k(ee_hbm, dst_hbm, zero_hbm, gd_hbm, idx_v, val_v, acc_sh, sem):
        sid = lax.axis_index("s")
        cid = lax.axis_index("c")

        @pl.when(sid == 0)
        def _():
            pltpu.sync_copy(zero_hbm, acc_sh)

        plsc.subcore_barrier()

        def accum(g, carry):
            r0 = pl.multiple_of(sid * (per_core // CH) + g * kb, kb)
            pltpu.sync_copy(dst_hbm.at[pl.ds(r0, kb)], idx_v)
            pltpu.sync_copy(ee_hbm.at[pl.ds(r0, kb)], val_v)
            for j in range(kb):
                pltpu.sync_copy(val_v.at[j], acc_sh.at[idx_v.at[j]], add=True)
            return carry

        lax.fori_loop(0, n2, accum, 0)
        plsc.subcore_barrier()
        wid = sid * NC + cid

        def readout(g, carry):
            r0 = pl.multiple_of(wid * (per_w // CH) + g * kb, kb)
            pltpu.sync_copy(dst_hbm.at[pl.ds(r0, kb)], idx_v)
            cps = [
                pltpu.async_copy(acc_sh.at[idx_v.at[j]], val_v.at[j], sem)
                for j in range(kb)
            ]
            for c in cps:
                c.wait()
            pltpu.sync_copy(val_v, gd_hbm.at[pl.ds(r0, kb)])
            return carry

        lax.fori_loop(0, n3, readout, 0)

    return k(ee3, dst2, zeros).reshape(b, w)


def _scatter_add(values, idx, n_out, zeros):
    b, w = values.shape
    kb = 1 if w >= 128 else _kb(w)
    blk = kb * CH
    per_w = b // NW
    assert per_w % blk == 0
    n_blocks = per_w // blk
    rows_per_tile = (n_out // 8 // NS) * 8
    rows_rem = n_out - rows_per_tile * NS
    idx2 = idx.reshape(-1, CH)
    val3 = values.reshape(-1, CH, w)
    cp = None if w % 128 == 0 else pltpu.CompilerParams(use_tc_tiling_on_sc=False)

    @functools.partial(
        pl.kernel,
        out_type=jax.ShapeDtypeStruct((2, n_out, w), jnp.float32),
        mesh=_sc_mesh(),
        scratch_types=[
            pltpu.VMEM((kb, CH), jnp.int32),
            pltpu.VMEM((kb, CH, w), jnp.float32),
            pltpu.VMEM_SHARED((n_out, w), jnp.float32),
        ],
        compiler_params=cp,
    )
    def k(val_hbm, idx_hbm, zero_hbm, out_hbm, idx_v, val_v, acc_sh):
        cid = lax.axis_index("c")
        sid = lax.axis_index("s")
        wid = sid * NC + cid

        @pl.when(sid == 0)
        def _():
            pltpu.sync_copy(zero_hbm, acc_sh)

        plsc.subcore_barrier()

        def body(g, carry):
            r0 = pl.multiple_of(wid * (per_w // CH) + g * kb, kb)
            pltpu.sync_copy(idx_hbm.at[pl.ds(r0, kb)], idx_v)
            pltpu.sync_copy(val_hbm.at[pl.ds(r0, kb)], val_v)
            for j in range(kb):
                pltpu.sync_copy(val_v.at[j], acc_sh.at[idx_v.at[j]], add=True)
            return carry

        lax.fori_loop(0, n_blocks, body, 0)
        plsc.subcore_barrier()
        r0 = pl.multiple_of(sid * rows_per_tile, 8)
        pltpu.sync_copy(
            acc_sh.at[pl.ds(r0, rows_per_tile)],
            out_hbm.at[cid, pl.ds(r0, rows_per_tile)],
        )
        if rows_rem:
            @pl.when(sid == 0)
            def _():
                rr = rows_per_tile * NS
                pltpu.sync_copy(
                    acc_sh.at[pl.ds(rr, rows_rem)],
                    out_hbm.at[cid, pl.ds(rr, rows_rem)],
                )

    parts = k(val3, idx2, zeros)
    return _add2(parts[0], parts[1])


def _head_mats(oe, on, attn):
    eye_e = jnp.eye(oe, dtype=jnp.float32)
    sum_e = jnp.tile(eye_e, (H, 1))
    attn_mat = jnp.zeros((H * oe, 16), jnp.float32)
    for h in range(H):
        attn_mat = attn_mat.at[h * oe:(h + 1) * oe, h].set(attn[0, h, :])
    expand = jnp.zeros((16, H * on), jnp.float32)
    for h in range(H):
        expand = expand.at[h, h * on:(h + 1) * on].set(1.0)
    sum_n = jnp.tile(jnp.eye(on, dtype=jnp.float32), (H, 1))
    return attn_mat, sum_e, expand, sum_n


def _egat(nfeats, efeats_raw, src_p, dst_p, num_nodes, p, out_n, out_e, n_real,
          need_nf=True):
    xs, xd = _gather2(nfeats, src_p, dst_p)
    attn_mat, sum_e, expand, sum_n = _head_mats(out_e, out_n, p["attn"])
    w_cat = jnp.concatenate([p["Wni"].T, p["Wnj"].T, p["Wfij"].T], axis=0)
    ef, ee = _edge_logits(
        xs, xd, efeats_raw, w_cat, p["bias"], attn_mat, sum_e, out_e, n_real,
    )
    if not need_nf:
        return None, ef
    zeros16 = jnp.zeros((num_nodes, 16), jnp.float32)
    gd = _softmax_denom(ee, dst_p, num_nodes, zeros16)
    msg = _edge_messages(xs, ee, gd, p["Wnode"].T, p["bnode"], expand, sum_n,
                         out_n)
    zeros_n = jnp.zeros((num_nodes, out_n), jnp.float32)
    nf = _scatter_add(msg, dst_p, num_nodes, zeros_n)
    return nf, ef


def _pad_edges(x, ep):
    e = x.shape[0]
    if e == ep:
        return x
    pad = [(0, ep - e)] + [(0, 0)] * (x.ndim - 1)
    return jnp.pad(x, pad)


def _mm_pairs(ef_p, w, b, n_rows):
    ke = ef_p.shape[1]
    n = w.shape[0]
    wt = w.T
    bm = _pick_bm(n_rows, cap=4096)

    x11 = ef_p.reshape(-1, 2 * ke)

    def body(x_ref, w_ref, b_ref, o_ref):
        o_ref[...] = (
            jnp.dot(x_ref[...], w_ref[...], preferred_element_type=jnp.float32)
            + b_ref[...]
        )

    return pl.pallas_call(
        body,
        grid=(n_rows // bm,),
        in_specs=[
            pl.BlockSpec((bm, 2 * ke), lambda i: (i, 0)),
            pl.BlockSpec((2 * ke, n), lambda i: (0, 0)),
            pl.BlockSpec((1, n), lambda i: (0, 0)),
        ],
        out_specs=pl.BlockSpec((bm, n), lambda i: (i, 0)),
        out_shape=jax.ShapeDtypeStruct((n_rows, n), jnp.float32),
    )(x11, wt, b.reshape(1, n))


def kernel(node_feats, edge_feats, node_path, edge_path, params,
           edge_index_lg, edge_index_gg):
    del node_path
    n1, e1 = node_feats.shape[0], edge_feats.shape[0]
    n2, e2 = e1 // 2, edge_path.shape[0]
    e1p = ((e1 + ALIGN - 1) // ALIGN) * ALIGN
    e2p = ((e2 + ALIGN - 1) // ALIGN) * ALIGN

    src1 = _pad_edges(edge_index_lg[0], e1p)
    dst1 = _pad_edges(edge_index_lg[1], e1p)
    src2 = _pad_edges(edge_index_gg[0], e2p)
    dst2 = _pad_edges(edge_index_gg[1], e2p)

    p1, p2 = params["gcn3"]
    nf1, ef = _egat(node_feats, edge_feats, src1, dst1, n1, p1, 128, 16, e1)
    _, ef = _egat(nf1, ef, src1, dst1, n1, p2, 128, 16, e1, need_nf=False)

    npth = _mm_pairs(ef, params["Wlin"], params["blin"], n2)

    h3, f3 = _egat(npth, edge_path, src2, dst2, n2, params["l4"], 16, 16, e2)
    return h3, f3[:e2]
